# baseline (speedup 1.0000x reference)
import functools

import jax
import jax.numpy as jnp
from jax import lax
from jax.experimental import pallas as pl
from jax.experimental.pallas import tpu as pltpu
from jax.experimental.pallas import tpu_sc as plsc

N = 3200000
NUM_SHOWERS = 512
NUM_EVENTS = 8
NC = 2
NS = 16
HALF = N // NC
TB = NUM_SHOWERS * NUM_EVENTS
TR = TB // 16

CS1 = 160000
NCH1 = HALF // CS1
SL1 = CS1 // NS
NV1 = SL1 // 16

CS2 = 160000
NCH2 = HALF // CS2
SL2 = CS2 // NS
NV2 = SL2 // 16
UNR = 5

_mesh = plsc.VectorSubcoreMesh(core_axis_name="c", subcore_axis_name="s")


def _seg_of(ivec, rs_rows):
    one = jnp.ones((16,), jnp.int32)
    zero = jnp.zeros((16,), jnp.int32)
    seg = jnp.where(ivec >= rs_rows[0], one, zero)
    for j in range(1, NUM_EVENTS - 1):
        seg = seg + jnp.where(ivec >= rs_rows[j], one, zero)
    return seg


@functools.partial(
    pl.kernel,
    out_type=jax.ShapeDtypeStruct((NC, TB), jnp.float32),
    mesh=_mesh,
    compiler_params=pltpu.CompilerParams(needs_layout_passes=False),
    scratch_types=[
        pltpu.VMEM((SL1,), jnp.int32),
        pltpu.VMEM((SL1,), jnp.int32),
        pltpu.VMEM((SL1,), jnp.float32),
        pltpu.VMEM((SL1,), jnp.float32),
        pltpu.VMEM((SL1,), jnp.float32),
        pltpu.VMEM((SL1,), jnp.float32),
        pltpu.VMEM((TB,), jnp.float32),
        pltpu.VMEM((NUM_EVENTS, 16), jnp.int32),
        pltpu.VMEM((TB // NS,), jnp.float32),
        pltpu.VMEM((TB // NS,), jnp.float32),
        pltpu.VMEM_SHARED((CS1,), jnp.int32),
        pltpu.VMEM_SHARED((CS1,), jnp.int32),
        pltpu.VMEM_SHARED((CS1,), jnp.float32),
        pltpu.VMEM_SHARED((CS1,), jnp.float32),
        pltpu.VMEM_SHARED((CS1,), jnp.float32),
        pltpu.VMEM_SHARED((CS1,), jnp.float32),
        pltpu.VMEM_SHARED((NS, TB), jnp.float32),
        pltpu.SemaphoreType.DMA,
        pltpu.SemaphoreType.DMA,
        pltpu.SemaphoreType.DMA,
        pltpu.SemaphoreType.DMA,
        pltpu.SemaphoreType.DMA,
        pltpu.SemaphoreType.DMA,
        pltpu.SemaphoreType.DMA,
        pltpu.SemaphoreType.DMA,
        pltpu.SemaphoreType.DMA,
        pltpu.SemaphoreType.DMA,
        pltpu.SemaphoreType.DMA,
        pltpu.SemaphoreType.DMA,
    ],
)
def _scatter_pass(sid_h, pcf_h, nrg_h, rsb_h, part_h,
                  tsid0, tsid1, tpcf0, tpcf1, tnrg0, tnrg1,
                  tbl, rs_v, stage, acc,
                  ssid0, ssid1, spcf0, spcf1, snrg0, snrg1, sharedtbl,
                  d00, d01, d02, d10, d11, d12,
                  x00, x01, x02, x10, x11, x12):
    c = lax.axis_index("c")
    s = lax.axis_index("s")
    cbase0 = pl.multiple_of(c * HALF, CS1)

    dsem = [(d00, d01, d02), (d10, d11, d12)]
    xsem = [(x00, x01, x02), (x10, x11, x12)]
    tbufs = [(tsid0, tpcf0, tnrg0), (tsid1, tpcf1, tnrg1)]
    sbufs = [(ssid0, spcf0, snrg0), (ssid1, spcf1, snrg1)]

    def _dma_args(i):
        slot = i % 2
        off = pl.multiple_of(cbase0 + i * CS1, CS1)
        ds_ = dsem[slot]
        ss, sp, sn = sbufs[slot]
        return [(sid_h.at[pl.ds(off, CS1)], ss, ds_[0]),
                (pcf_h.at[pl.ds(off, CS1)], sp, ds_[1]),
                (nrg_h.at[pl.ds(off, CS1)], sn, ds_[2])]

    def _dma_go(i):
        @pl.when(s == 0)
        def _():
            for a in _dma_args(i):
                pltpu.async_copy(*a)

    def _dma_wait(i):
        @pl.when(s == 0)
        def _():
            for a in _dma_args(i):
                pltpu.make_async_copy(*a).wait()

    def _xbar_go(i):
        slot = i % 2
        so = pl.multiple_of(s * SL1, 8)
        xs = xsem[slot]
        tb_ = tbufs[slot]
        ss, sp, sn = sbufs[slot]
        return [pltpu.async_copy(ss.at[pl.ds(so, SL1)], tb_[0], xs[0]),
                pltpu.async_copy(sp.at[pl.ds(so, SL1)], tb_[1], xs[1]),
                pltpu.async_copy(sn.at[pl.ds(so, SL1)], tb_[2], xs[2])]

    _dma_go(0)

    pltpu.sync_copy(rsb_h, rs_v)
    zf = jnp.zeros((16,), jnp.float32)

    def _zero(i, carry):
        tbl[pl.ds(i * 16, 16)] = zf
        return carry

    lax.fori_loop(0, TR, _zero, 0)

    rs_rows = [rs_v[j] for j in range(NUM_EVENTS - 1)]
    iota = lax.iota(jnp.int32, 16)

    _dma_wait(0)
    plsc.subcore_barrier()
    xh = {0: _xbar_go(0)}
    _dma_go(1)

    for i in range(NCH1):
        for h in xh.pop(i):
            h.wait()
        plsc.subcore_barrier()
        if i + 2 < NCH1:
            _dma_go(i + 2)
        if i + 1 < NCH1:
            _dma_wait(i + 1)
            plsc.subcore_barrier()
            xh[i + 1] = _xbar_go(i + 1)

        sb, pb, nb = tbufs[i % 2]
        cbase = cbase0 + i * CS1 + s * SL1
        seg_lo = _seg_of(jnp.full((16,), cbase, jnp.int32), rs_rows)
        seg_hi = _seg_of(jnp.full((16,), cbase + (SL1 - 1), jnp.int32), rs_rows)
        segbase = 1 + (seg_lo << 9)

        def _fast(carry):
            def _body(k, c2):
                for u in range(UNR):
                    off = (k * UNR + u) * 16
                    sid = sb[pl.ds(off, 16)]
                    pcf = pb[pl.ds(off, 16)]
                    nrg = nb[pl.ds(off, 16)]
                    idx = sid + segbase
                    val = nrg * jnp.where(sid >= 0, pcf, zf)
                    plsc.addupdate_scatter(tbl, [idx], val)
                return c2
            return lax.fori_loop(0, NV1 // UNR, _body, carry)

        def _slow(carry):
            def _body(k, c2):
                for u in range(UNR):
                    off = (k * UNR + u) * 16
                    sid = sb[pl.ds(off, 16)]
                    pcf = pb[pl.ds(off, 16)]
                    nrg = nb[pl.ds(off, 16)]
                    ivec = cbase + off + iota
                    seg = _seg_of(ivec, rs_rows)
                    idx = sid + 1 + (seg << 9)
                    val = nrg * jnp.where(sid >= 0, pcf, zf)
                    plsc.addupdate_scatter(tbl, [idx], val)
                return c2
            return lax.fori_loop(0, NV1 // UNR, _body, carry)

        lax.cond(jnp.max(seg_hi) == jnp.max(seg_lo), _fast, _slow, 0)

    pltpu.sync_copy(tbl, sharedtbl.at[s])
    plsc.subcore_barrier()
    blk = TB // NS
    rbase = s * blk
    pltpu.sync_copy(sharedtbl.at[0, pl.ds(rbase, blk)], acc)
    for t in range(1, NS):
        pltpu.sync_copy(sharedtbl.at[t, pl.ds(rbase, blk)], stage)
        for r in range(blk // 16):
            acc[pl.ds(r * 16, 16)] = acc[pl.ds(r * 16, 16)] + stage[pl.ds(r * 16, 16)]
    pltpu.sync_copy(acc, part_h.at[c, pl.ds(rbase, blk)])


@functools.partial(
    pl.kernel,
    out_type=jax.ShapeDtypeStruct((N,), jnp.float32),
    mesh=_mesh,
    compiler_params=pltpu.CompilerParams(needs_layout_passes=False),
    scratch_types=[
        pltpu.VMEM((SL2,), jnp.int32),
        pltpu.VMEM((SL2,), jnp.int32),
        pltpu.VMEM((SL2,), jnp.float32),
        pltpu.VMEM((SL2,), jnp.float32),
        pltpu.VMEM((TB,), jnp.float32),
        pltpu.VMEM((TB,), jnp.float32),
        pltpu.VMEM((NUM_EVENTS, 16), jnp.int32),
        pltpu.VMEM_SHARED((CS2,), jnp.int32),
        pltpu.VMEM_SHARED((CS2,), jnp.int32),
        pltpu.VMEM_SHARED((CS2,), jnp.float32),
        pltpu.VMEM_SHARED((CS2,), jnp.float32),
        pltpu.SemaphoreType.DMA,
        pltpu.SemaphoreType.DMA,
        pltpu.SemaphoreType.DMA,
        pltpu.SemaphoreType.DMA,
        pltpu.SemaphoreType.DMA,
        pltpu.SemaphoreType.DMA,
        pltpu.SemaphoreType.DMA,
        pltpu.SemaphoreType.DMA,
    ],
)
def _gather_pass(sid_h, rsb_h, part_h, out_h,
                 tsid0, tsid1, tout0, tout1, tblA, tblB, rs_v,
                 ssid0, ssid1, sout0, sout1,
                 di0, di1, xi0, xi1, xo0, xo1, do0, do1):
    c = lax.axis_index("c")
    s = lax.axis_index("s")
    cbase0 = pl.multiple_of(c * HALF, CS2)

    disem = [di0, di1]
    xisem = [xi0, xi1]
    xosem = [xo0, xo1]
    dosem = [do0, do1]
    tins = [tsid0, tsid1]
    touts = [tout0, tout1]
    sins = [ssid0, ssid1]
    souts = [sout0, sout1]

    def _din_args(i):
        slot = i % 2
        off = pl.multiple_of(cbase0 + i * CS2, CS2)
        return (sid_h.at[pl.ds(off, CS2)], sins[slot], disem[slot])

    def _dout_args(i):
        slot = i % 2
        off = pl.multiple_of(cbase0 + i * CS2, CS2)
        return (souts[slot], out_h.at[pl.ds(off, CS2)], dosem[slot])

    def _din_go(i):
        @pl.when(s == 0)
        def _():
            pltpu.async_copy(*_din_args(i))

    def _din_wait(i):
        @pl.when(s == 0)
        def _():
            pltpu.make_async_copy(*_din_args(i)).wait()

    def _dout_go(i):
        @pl.when(s == 0)
        def _():
            pltpu.async_copy(*_dout_args(i))

    def _dout_wait(i):
        @pl.when(s == 0)
        def _():
            pltpu.make_async_copy(*_dout_args(i)).wait()

    def _xin_go(i):
        slot = i % 2
        so = pl.multiple_of(s * SL2, 8)
        return pltpu.async_copy(sins[slot].at[pl.ds(so, SL2)], tins[slot],
                                xisem[slot])

    def _xout(i):
        slot = i % 2
        so = pl.multiple_of(s * SL2, 8)
        pltpu.async_copy(touts[slot], souts[slot].at[pl.ds(so, SL2)],
                         xosem[slot]).wait()

    _din_go(0)

    pltpu.sync_copy(rsb_h, rs_v)
    pltpu.sync_copy(part_h.at[0], tblA)
    pltpu.sync_copy(part_h.at[1], tblB)

    def _combine(i, carry):
        tblA[pl.ds(i * 16, 16)] = tblA[pl.ds(i * 16, 16)] + tblB[pl.ds(i * 16, 16)]
        return carry

    lax.fori_loop(0, TR, _combine, 0)

    rs_rows = [rs_v[j] for j in range(NUM_EVENTS - 1)]
    iota = lax.iota(jnp.int32, 16)

    _din_wait(0)
    plsc.subcore_barrier()
    xh = {0: _xin_go(0)}
    _din_go(1)

    for i in range(NCH2):
        xh.pop(i).wait()
        plsc.subcore_barrier()
        if i + 2 < NCH2:
            _din_go(i + 2)
        if i + 1 < NCH2:
            _din_wait(i + 1)
            plsc.subcore_barrier()
            xh[i + 1] = _xin_go(i + 1)

        sb = tins[i % 2]
        ob = touts[i % 2]
        cbase = cbase0 + i * CS2 + s * SL2
        seg_lo = _seg_of(jnp.full((16,), cbase, jnp.int32), rs_rows)
        seg_hi = _seg_of(jnp.full((16,), cbase + (SL2 - 1), jnp.int32), rs_rows)
        segbase = 1 + (seg_lo << 9)

        def _fast(carry):
            def _body(k, c2):
                for u in range(UNR):
                    off = (k * UNR + u) * 16
                    sid = sb[pl.ds(off, 16)]
                    idx = sid + segbase
                    ob[pl.ds(off, 16)] = plsc.load_gather(tblA, [idx])
                return c2
            return lax.fori_loop(0, NV2 // UNR, _body, carry)

        def _slow(carry):
            def _body(k, c2):
                for u in range(UNR):
                    off = (k * UNR + u) * 16
                    sid = sb[pl.ds(off, 16)]
                    ivec = cbase + off + iota
                    seg = _seg_of(ivec, rs_rows)
                    idx = sid + 1 + (seg << 9)
                    ob[pl.ds(off, 16)] = plsc.load_gather(tblA, [idx])
                return c2
            return lax.fori_loop(0, NV2 // UNR, _body, carry)

        lax.cond(jnp.max(seg_hi) == jnp.max(seg_lo), _fast, _slow, 0)

        if i >= 2:
            _dout_wait(i - 2)
        plsc.subcore_barrier()
        _xout(i)
        plsc.subcore_barrier()
        _dout_go(i)

    for i in (NCH2 - 2, NCH2 - 1):
        if i >= 0:
            _dout_wait(i)
    plsc.subcore_barrier()


def kernel(pred_sid, pred_corr_factor, rechit_energy, row_splits):
    sid = pred_sid[:, 0]
    pcf = pred_corr_factor[:, 0]
    nrg = rechit_energy[:, 0]
    rs_inner = row_splits[1:NUM_EVENTS].astype(jnp.int32)
    rsb = jnp.concatenate(
        [jnp.broadcast_to(rs_inner[:, None], (NUM_EVENTS - 1, 16)),
         jnp.full((1, 16), jnp.int32(0x7FFFFFFF))], axis=0)
    parts = _scatter_pass(sid, pcf, nrg, rsb)
    out = _gather_pass(sid, rsb, parts)
    return out[:, None]


# speedup vs baseline: 1.3253x; 1.3253x over previous
import functools

import jax
import jax.numpy as jnp
from jax import lax
from jax.experimental import pallas as pl
from jax.experimental.pallas import tpu as pltpu
from jax.experimental.pallas import tpu_sc as plsc

N = 3200000
NUM_SHOWERS = 512
NUM_EVENTS = 8
NC = 2
NS = 16
NW = NC * NS
C = N // NW
CH = 20000
NCHUNK = C // CH
NV = CH // 16
TB = NUM_SHOWERS * NUM_EVENTS
TR = TB // 16

_mesh = plsc.VectorSubcoreMesh(core_axis_name="c", subcore_axis_name="s")


def _worker(c, s):
    return s * NC + c


def _seg_of(ivec, rs_rows):
    one = jnp.ones((16,), jnp.int32)
    zero = jnp.zeros((16,), jnp.int32)
    seg = jnp.where(ivec >= rs_rows[0], one, zero)
    for j in range(1, NUM_EVENTS - 1):
        seg = seg + jnp.where(ivec >= rs_rows[j], one, zero)
    return seg


@functools.partial(
    pl.kernel,
    out_type=jax.ShapeDtypeStruct((NC, TB), jnp.float32),
    mesh=_mesh,
    compiler_params=pltpu.CompilerParams(needs_layout_passes=False),
    scratch_types=[
        pltpu.VMEM((CH,), jnp.int32),
        pltpu.VMEM((CH,), jnp.int32),
        pltpu.VMEM((CH,), jnp.float32),
        pltpu.VMEM((CH,), jnp.float32),
        pltpu.VMEM((CH,), jnp.float32),
        pltpu.VMEM((CH,), jnp.float32),
        pltpu.VMEM((TB,), jnp.float32),
        pltpu.VMEM((NUM_EVENTS, 16), jnp.int32),
        pltpu.VMEM((TB // NS,), jnp.float32),
        pltpu.VMEM((TB // NS,), jnp.float32),
        pltpu.VMEM_SHARED((NS, TB), jnp.float32),
        pltpu.SemaphoreType.DMA,
        pltpu.SemaphoreType.DMA,
        pltpu.SemaphoreType.DMA,
        pltpu.SemaphoreType.DMA,
        pltpu.SemaphoreType.DMA,
        pltpu.SemaphoreType.DMA,
    ],
)
def _scatter_pass(sid_h, pcf_h, nrg_h, rsb_h, part_h,
                  sid0, sid1, pcf0, pcf1, nrg0, nrg1,
                  tbl, rs_v, stage, acc, shared,
                  ss0, ss1, sp0, sp1, sn0, sn1):
    c = lax.axis_index("c")
    s = lax.axis_index("s")
    base = pl.multiple_of(_worker(c, s) * C, CH)

    pltpu.sync_copy(rsb_h, rs_v)

    zf = jnp.zeros((16,), jnp.float32)

    def _zero(i, carry):
        tbl[pl.ds(i * 16, 16)] = zf
        return carry

    lax.fori_loop(0, TR, _zero, 0)

    rs_rows = [rs_v[j] for j in range(NUM_EVENTS - 1)]
    iota = lax.iota(jnp.int32, 16)

    bufs = [(sid0, pcf0, nrg0, ss0, sp0, sn0),
            (sid1, pcf1, nrg1, ss1, sp1, sn1)]

    def _start(ci):
        sb, pb, nb, s_s, s_p, s_n = bufs[ci % 2]
        off = pl.multiple_of(base + ci * CH, CH)
        return (pltpu.async_copy(sid_h.at[pl.ds(off, CH)], sb, s_s),
                pltpu.async_copy(pcf_h.at[pl.ds(off, CH)], pb, s_p),
                pltpu.async_copy(nrg_h.at[pl.ds(off, CH)], nb, s_n))

    pending = _start(0)
    for ci in range(NCHUNK):
        for h in pending:
            h.wait()
        if ci + 1 < NCHUNK:
            pending = _start(ci + 1)
        sb, pb, nb = bufs[ci % 2][:3]
        cbase = base + ci * CH
        seg_lo = _seg_of(jnp.full((16,), cbase, jnp.int32), rs_rows)
        seg_hi = _seg_of(jnp.full((16,), cbase + (CH - 1), jnp.int32), rs_rows)
        segbase = 1 + (seg_lo << 9)

        def _fast(carry):
            def _body(i, c2):
                off = i * 16
                sid = sb[pl.ds(off, 16)]
                pcf = pb[pl.ds(off, 16)]
                nrg = nb[pl.ds(off, 16)]
                idx = sid + segbase
                val = nrg * jnp.where(sid >= 0, pcf, zf)
                plsc.addupdate_scatter(tbl, [idx], val)
                return c2
            return lax.fori_loop(0, NV, _body, carry)

        def _slow(carry):
            def _body(i, c2):
                off = i * 16
                sid = sb[pl.ds(off, 16)]
                pcf = pb[pl.ds(off, 16)]
                nrg = nb[pl.ds(off, 16)]
                ivec = cbase + off + iota
                seg = _seg_of(ivec, rs_rows)
                idx = sid + 1 + (seg << 9)
                val = nrg * jnp.where(sid >= 0, pcf, zf)
                plsc.addupdate_scatter(tbl, [idx], val)
                return c2
            return lax.fori_loop(0, NV, _body, carry)

        lax.cond(jnp.max(seg_hi) == jnp.max(seg_lo), _fast, _slow, 0)

    pltpu.sync_copy(tbl, shared.at[s])
    plsc.subcore_barrier()
    blk = TB // NS
    rbase = s * blk
    pltpu.sync_copy(shared.at[0, pl.ds(rbase, blk)], acc)
    for t in range(1, NS):
        pltpu.sync_copy(shared.at[t, pl.ds(rbase, blk)], stage)
        for r in range(blk // 16):
            acc[pl.ds(r * 16, 16)] = acc[pl.ds(r * 16, 16)] + stage[pl.ds(r * 16, 16)]
    pltpu.sync_copy(acc, part_h.at[c, pl.ds(rbase, blk)])


@functools.partial(
    pl.kernel,
    out_type=jax.ShapeDtypeStruct((N,), jnp.float32),
    mesh=_mesh,
    compiler_params=pltpu.CompilerParams(needs_layout_passes=False),
    scratch_types=[
        pltpu.VMEM((CH,), jnp.int32),
        pltpu.VMEM((CH,), jnp.int32),
        pltpu.VMEM((CH,), jnp.float32),
        pltpu.VMEM((CH,), jnp.float32),
        pltpu.VMEM((TB,), jnp.float32),
        pltpu.VMEM((TB,), jnp.float32),
        pltpu.VMEM((NUM_EVENTS, 16), jnp.int32),
        pltpu.SemaphoreType.DMA,
        pltpu.SemaphoreType.DMA,
        pltpu.SemaphoreType.DMA,
        pltpu.SemaphoreType.DMA,
    ],
)
def _gather_pass(sid_h, rsb_h, part_h, out_h,
                 sid0, sid1, outb0, outb1, tblA, tblB, rs_v,
                 ss0, ss1, so0, so1):
    c = lax.axis_index("c")
    s = lax.axis_index("s")
    base = pl.multiple_of(_worker(c, s) * C, CH)

    pltpu.sync_copy(rsb_h, rs_v)
    pltpu.sync_copy(part_h.at[0], tblA)
    pltpu.sync_copy(part_h.at[1], tblB)

    def _combine(i, carry):
        tblA[pl.ds(i * 16, 16)] = tblA[pl.ds(i * 16, 16)] + tblB[pl.ds(i * 16, 16)]
        return carry

    lax.fori_loop(0, TR, _combine, 0)

    rs_rows = [rs_v[j] for j in range(NUM_EVENTS - 1)]
    iota = lax.iota(jnp.int32, 16)

    ins = [(sid0, ss0), (sid1, ss1)]
    outs = [(outb0, so0), (outb1, so1)]

    def _start_in(ci):
        sb, s_s = ins[ci % 2]
        off = pl.multiple_of(base + ci * CH, CH)
        return pltpu.async_copy(sid_h.at[pl.ds(off, CH)], sb, s_s)

    def _start_out(ci):
        ob, s_o = outs[ci % 2]
        off = pl.multiple_of(base + ci * CH, CH)
        return pltpu.async_copy(ob, out_h.at[pl.ds(off, CH)], s_o)

    pend_in = _start_in(0)
    pend_out = {}
    for ci in range(NCHUNK):
        pend_in.wait()
        if ci + 1 < NCHUNK:
            pend_in = _start_in(ci + 1)
        if ci - 2 in pend_out:
            pend_out.pop(ci - 2).wait()
        sb = ins[ci % 2][0]
        ob = outs[ci % 2][0]
        cbase = base + ci * CH
        seg_lo = _seg_of(jnp.full((16,), cbase, jnp.int32), rs_rows)
        seg_hi = _seg_of(jnp.full((16,), cbase + (CH - 1), jnp.int32), rs_rows)
        segbase = 1 + (seg_lo << 9)

        def _fast(carry):
            def _body(i, c2):
                off = i * 16
                sid = sb[pl.ds(off, 16)]
                idx = sid + segbase
                ob[pl.ds(off, 16)] = plsc.load_gather(tblA, [idx])
                return c2
            return lax.fori_loop(0, NV, _body, carry)

        def _slow(carry):
            def _body(i, c2):
                off = i * 16
                sid = sb[pl.ds(off, 16)]
                ivec = cbase + off + iota
                seg = _seg_of(ivec, rs_rows)
                idx = sid + 1 + (seg << 9)
                ob[pl.ds(off, 16)] = plsc.load_gather(tblA, [idx])
                return c2
            return lax.fori_loop(0, NV, _body, carry)

        lax.cond(jnp.max(seg_hi) == jnp.max(seg_lo), _fast, _slow, 0)
        pend_out[ci] = _start_out(ci)

    for ci in sorted(pend_out):
        pend_out[ci].wait()


def kernel(pred_sid, pred_corr_factor, rechit_energy, row_splits):
    sid = pred_sid[:, 0]
    pcf = pred_corr_factor[:, 0]
    nrg = rechit_energy[:, 0]
    rs_inner = row_splits[1:NUM_EVENTS].astype(jnp.int32)
    rsb = jnp.concatenate(
        [jnp.broadcast_to(rs_inner[:, None], (NUM_EVENTS - 1, 16)),
         jnp.full((1, 16), jnp.int32(0x7FFFFFFF))], axis=0)
    parts = _scatter_pass(sid, pcf, nrg, rsb)
    out = _gather_pass(sid, rsb, parts)
    return out[:, None]


# speedup vs baseline: 1.3455x; 1.0152x over previous
import functools

import jax
import jax.numpy as jnp
from jax import lax
from jax.experimental import pallas as pl
from jax.experimental.pallas import tpu as pltpu
from jax.experimental.pallas import tpu_sc as plsc

N = 3200000
NUM_SHOWERS = 512
NUM_EVENTS = 8
NC = 2
NS = 16
NW = NC * NS
C = N // NW
CH = 20000
NCHUNK = C // CH
NV = CH // 16
TB = NUM_SHOWERS * NUM_EVENTS
TR = TB // 16

_mesh = plsc.VectorSubcoreMesh(core_axis_name="c", subcore_axis_name="s")


def _worker(c, s):
    return s * NC + c


def _seg_of(ivec, rs_rows):
    one = jnp.ones((16,), jnp.int32)
    zero = jnp.zeros((16,), jnp.int32)
    seg = jnp.where(ivec >= rs_rows[0], one, zero)
    for j in range(1, NUM_EVENTS - 1):
        seg = seg + jnp.where(ivec >= rs_rows[j], one, zero)
    return seg


@functools.partial(
    pl.kernel,
    out_type=jax.ShapeDtypeStruct((NC, TB), jnp.float32),
    mesh=_mesh,
    compiler_params=pltpu.CompilerParams(needs_layout_passes=False),
    scratch_types=[
        pltpu.VMEM((CH,), jnp.int32),
        pltpu.VMEM((CH,), jnp.int32),
        pltpu.VMEM((CH,), jnp.float32),
        pltpu.VMEM((CH,), jnp.float32),
        pltpu.VMEM((CH,), jnp.float32),
        pltpu.VMEM((CH,), jnp.float32),
        pltpu.VMEM((TB,), jnp.float32),
        pltpu.VMEM((16,), jnp.int32),
        pltpu.VMEM((TB // NS,), jnp.float32),
        pltpu.VMEM((TB // NS,), jnp.float32),
        pltpu.VMEM_SHARED((NS, TB), jnp.float32),
        pltpu.SemaphoreType.DMA,
        pltpu.SemaphoreType.DMA,
        pltpu.SemaphoreType.DMA,
        pltpu.SemaphoreType.DMA,
        pltpu.SemaphoreType.DMA,
        pltpu.SemaphoreType.DMA,
    ],
)
def _scatter_pass(sid_h, pcf_h, nrg_h, rs_h, part_h,
                  sid0, sid1, pcf0, pcf1, nrg0, nrg1,
                  tbl, rs_v, stage, acc, shared,
                  ss0, ss1, sp0, sp1, sn0, sn1):
    c = lax.axis_index("c")
    s = lax.axis_index("s")
    base = pl.multiple_of(_worker(c, s) * C, CH)

    pltpu.sync_copy(rs_h, rs_v.at[pl.ds(0, NUM_EVENTS + 1)])

    zf = jnp.zeros((16,), jnp.float32)

    def _zero(i, carry):
        tbl[pl.ds(i * 16, 16)] = zf
        return carry

    lax.fori_loop(0, TR, _zero, 0)

    rs_all = rs_v[pl.ds(0, 16)]
    rs_rows = [jnp.full((16,), rs_all[j + 1], jnp.int32)
               for j in range(NUM_EVENTS - 1)]
    iota = lax.iota(jnp.int32, 16)

    bufs = [(sid0, pcf0, nrg0, ss0, sp0, sn0),
            (sid1, pcf1, nrg1, ss1, sp1, sn1)]

    def _start(ci):
        sb, pb, nb, s_s, s_p, s_n = bufs[ci % 2]
        off = pl.multiple_of(base + ci * CH, CH)
        return (pltpu.async_copy(sid_h.at[pl.ds(off, CH)], sb, s_s),
                pltpu.async_copy(pcf_h.at[pl.ds(off, CH)], pb, s_p),
                pltpu.async_copy(nrg_h.at[pl.ds(off, CH)], nb, s_n))

    pending = _start(0)
    for ci in range(NCHUNK):
        for h in pending:
            h.wait()
        if ci + 1 < NCHUNK:
            pending = _start(ci + 1)
        sb, pb, nb = bufs[ci % 2][:3]
        cbase = base + ci * CH
        seg_lo = _seg_of(jnp.full((16,), cbase, jnp.int32), rs_rows)
        seg_hi = _seg_of(jnp.full((16,), cbase + (CH - 1), jnp.int32), rs_rows)
        segbase = 1 + (seg_lo << 9)

        def _fast(carry):
            def _body(i, c2):
                off = i * 16
                sid = sb[pl.ds(off, 16)]
                pcf = pb[pl.ds(off, 16)]
                nrg = nb[pl.ds(off, 16)]
                idx = sid + segbase
                val = nrg * jnp.where(sid >= 0, pcf, zf)
                plsc.addupdate_scatter(tbl, [idx], val)
                return c2
            return lax.fori_loop(0, NV, _body, carry)

        def _slow(carry):
            def _body(i, c2):
                off = i * 16
                sid = sb[pl.ds(off, 16)]
                pcf = pb[pl.ds(off, 16)]
                nrg = nb[pl.ds(off, 16)]
                ivec = cbase + off + iota
                seg = _seg_of(ivec, rs_rows)
                idx = sid + 1 + (seg << 9)
                val = nrg * jnp.where(sid >= 0, pcf, zf)
                plsc.addupdate_scatter(tbl, [idx], val)
                return c2
            return lax.fori_loop(0, NV, _body, carry)

        lax.cond(jnp.max(seg_hi) == jnp.max(seg_lo), _fast, _slow, 0)

    pltpu.sync_copy(tbl, shared.at[s])
    plsc.subcore_barrier()
    blk = TB // NS
    rbase = s * blk
    pltpu.sync_copy(shared.at[0, pl.ds(rbase, blk)], acc)
    for t in range(1, NS):
        pltpu.sync_copy(shared.at[t, pl.ds(rbase, blk)], stage)
        for r in range(blk // 16):
            acc[pl.ds(r * 16, 16)] = acc[pl.ds(r * 16, 16)] + stage[pl.ds(r * 16, 16)]
    pltpu.sync_copy(acc, part_h.at[c, pl.ds(rbase, blk)])


@functools.partial(
    pl.kernel,
    out_type=jax.ShapeDtypeStruct((N,), jnp.float32),
    mesh=_mesh,
    compiler_params=pltpu.CompilerParams(needs_layout_passes=False),
    scratch_types=[
        pltpu.VMEM((CH,), jnp.int32),
        pltpu.VMEM((CH,), jnp.int32),
        pltpu.VMEM((CH,), jnp.float32),
        pltpu.VMEM((CH,), jnp.float32),
        pltpu.VMEM((TB,), jnp.float32),
        pltpu.VMEM((TB,), jnp.float32),
        pltpu.VMEM((16,), jnp.int32),
        pltpu.SemaphoreType.DMA,
        pltpu.SemaphoreType.DMA,
        pltpu.SemaphoreType.DMA,
        pltpu.SemaphoreType.DMA,
    ],
)
def _gather_pass(sid_h, rs_h, part_h, out_h,
                 sid0, sid1, outb0, outb1, tblA, tblB, rs_v,
                 ss0, ss1, so0, so1):
    c = lax.axis_index("c")
    s = lax.axis_index("s")
    base = pl.multiple_of(_worker(c, s) * C, CH)

    pltpu.sync_copy(rs_h, rs_v.at[pl.ds(0, NUM_EVENTS + 1)])
    pltpu.sync_copy(part_h.at[0], tblA)
    pltpu.sync_copy(part_h.at[1], tblB)

    def _combine(i, carry):
        tblA[pl.ds(i * 16, 16)] = tblA[pl.ds(i * 16, 16)] + tblB[pl.ds(i * 16, 16)]
        return carry

    lax.fori_loop(0, TR, _combine, 0)

    rs_all = rs_v[pl.ds(0, 16)]
    rs_rows = [jnp.full((16,), rs_all[j + 1], jnp.int32)
               for j in range(NUM_EVENTS - 1)]
    iota = lax.iota(jnp.int32, 16)

    ins = [(sid0, ss0), (sid1, ss1)]
    outs = [(outb0, so0), (outb1, so1)]

    def _start_in(ci):
        sb, s_s = ins[ci % 2]
        off = pl.multiple_of(base + ci * CH, CH)
        return pltpu.async_copy(sid_h.at[pl.ds(off, CH)], sb, s_s)

    def _start_out(ci):
        ob, s_o = outs[ci % 2]
        off = pl.multiple_of(base + ci * CH, CH)
        return pltpu.async_copy(ob, out_h.at[pl.ds(off, CH)], s_o)

    pend_in = _start_in(0)
    pend_out = {}
    for ci in range(NCHUNK):
        pend_in.wait()
        if ci + 1 < NCHUNK:
            pend_in = _start_in(ci + 1)
        if ci - 2 in pend_out:
            pend_out.pop(ci - 2).wait()
        sb = ins[ci % 2][0]
        ob = outs[ci % 2][0]
        cbase = base + ci * CH
        seg_lo = _seg_of(jnp.full((16,), cbase, jnp.int32), rs_rows)
        seg_hi = _seg_of(jnp.full((16,), cbase + (CH - 1), jnp.int32), rs_rows)
        segbase = 1 + (seg_lo << 9)

        def _fast(carry):
            def _body(i, c2):
                off = i * 16
                sid = sb[pl.ds(off, 16)]
                idx = sid + segbase
                ob[pl.ds(off, 16)] = plsc.load_gather(tblA, [idx])
                return c2
            return lax.fori_loop(0, NV, _body, carry)

        def _slow(carry):
            def _body(i, c2):
                off = i * 16
                sid = sb[pl.ds(off, 16)]
                ivec = cbase + off + iota
                seg = _seg_of(ivec, rs_rows)
                idx = sid + 1 + (seg << 9)
                ob[pl.ds(off, 16)] = plsc.load_gather(tblA, [idx])
                return c2
            return lax.fori_loop(0, NV, _body, carry)

        lax.cond(jnp.max(seg_hi) == jnp.max(seg_lo), _fast, _slow, 0)
        pend_out[ci] = _start_out(ci)

    for ci in sorted(pend_out):
        pend_out[ci].wait()


def kernel(pred_sid, pred_corr_factor, rechit_energy, row_splits):
    sid = pred_sid[:, 0]
    pcf = pred_corr_factor[:, 0]
    nrg = rechit_energy[:, 0]
    parts = _scatter_pass(sid, pcf, nrg, row_splits)
    out = _gather_pass(sid, row_splits, parts)
    return out[:, None]
